# Initial kernel scaffold; baseline (speedup 1.0000x reference)
import jax
import jax.numpy as jnp
from jax.experimental import pallas as pl


def kernel(z, W0, W1, W2, W3):
    raise NotImplementedError("write your pallas kernel here")


# speedup vs baseline: 1.7507x; 1.7507x over previous
import functools

import jax
import jax.numpy as jnp
from jax.experimental import pallas as pl
from jax.experimental.pallas import tpu as pltpu
from jax.experimental.pallas import tpu_sc as plsc

NUM_LAYERS = 4
K = 8192
D = 256
N = 16384
TB = 256
GW = 128


_SEG1 = 2731
_SEG2 = 5462


def _layer_body(z_ref, *refs, mode, tail):
    if mode == 0:
        (wt_ref, c_ref, wb2t_ref, b2_ref) = refs
    elif mode == 1:
        (g_ref, wt_ref, r_ref, c_ref, wb2t_ref, b2_ref) = refs
    else:
        (q_ref, g_ref, wt_ref, qout_ref, r_ref, c_ref, wb2t_ref, b2_ref) = refs

    @pl.when(pl.program_id(0) == 0)
    def _():
        w = wt_ref[...]
        b2_ref[...] = jnp.sum(w * w, axis=0, keepdims=True)
        wb2t_ref[...] = (w * 2.0).astype(jnp.bfloat16)

    if mode == 0:
        r = z_ref[...]
    elif mode == 1:
        q = g_ref[...]
        r = z_ref[...] - q
        r_ref[...] = r
    else:
        q = q_ref[...] + g_ref[...]
        qout_ref[...] = q
        r = z_ref[...] - q
        r_ref[...] = r

    rb = r.astype(jnp.bfloat16)
    ab2 = jax.lax.dot_general(
        rb, wb2t_ref[...], (((1,), (0,)), ((), ())),
        preferred_element_type=jnp.float32)
    if tail == "fast":
        s = b2_ref[...] - ab2
        c = jnp.argmin(s, axis=1)
        c_ref[...] = c[:, None].astype(jnp.int32)
        return

    a2 = jnp.sum(r * r, axis=1, keepdims=True)
    d2 = (a2 + b2_ref[...]) - ab2
    dist = jnp.sqrt(jnp.maximum(d2, 0.0))
    iota = jax.lax.broadcasted_iota(jnp.int32, dist.shape, 1)
    accv = acci = None
    for (lo, hi) in [(0, _SEG1), (_SEG1, _SEG2), (_SEG2, K)]:
        mask = (iota >= lo) & (iota < hi)
        seg = jnp.where(mask, dist, jnp.inf)
        mv = jnp.min(seg, axis=1, keepdims=True)
        mi = jnp.min(jnp.where(mask & (seg == mv), iota, K), axis=1, keepdims=True)
        if accv is None:
            accv, acci = mv, mi
        else:
            take = (mv < accv) | ((mv == accv) & (mi < acci))
            accv = jnp.where(take, mv, accv)
            acci = jnp.where(take, mi, acci)
        accv = accv.astype(jnp.bfloat16).astype(jnp.float32)
    c_ref[...] = acci.astype(jnp.int32)


def _tc_layer(z, qprev, gprev, wt, *, mode, tail="fast"):
    grid = (N // TB,)
    tok = pl.BlockSpec((TB, D), lambda i: (i, 0))
    wspec = pl.BlockSpec((D, K), lambda i: (0, 0))
    cspec = pl.BlockSpec((TB, 1), lambda i: (i, 0))
    fdt = jax.ShapeDtypeStruct((N, D), jnp.float32)
    cdt = jax.ShapeDtypeStruct((N, 1), jnp.int32)
    scratch = [pltpu.VMEM((D, K), jnp.bfloat16), pltpu.VMEM((1, K), jnp.float32)]
    params = pltpu.CompilerParams(dimension_semantics=("arbitrary",))

    if mode == 0:
        return pl.pallas_call(
            functools.partial(_layer_body, mode=0, tail=tail),
            grid=grid,
            in_specs=[tok, wspec],
            out_specs=cspec,
            out_shape=cdt,
            scratch_shapes=scratch,
            compiler_params=params,
        )(z, wt)
    if mode == 1:
        return pl.pallas_call(
            functools.partial(_layer_body, mode=1, tail=tail),
            grid=grid,
            in_specs=[tok, tok, wspec],
            out_specs=(tok, cspec),
            out_shape=(fdt, cdt),
            scratch_shapes=scratch,
            compiler_params=params,
        )(z, gprev, wt)
    return pl.pallas_call(
        functools.partial(_layer_body, mode=2, tail=tail),
        grid=grid,
        in_specs=[tok, tok, tok, wspec],
        out_specs=(tok, tok, cspec),
        out_shape=(fdt, fdt, cdt),
        scratch_shapes=scratch,
        compiler_params=params,
    )(z, qprev, gprev, wt)


def _sc_gather(w, idx):
    mesh = plsc.VectorSubcoreMesh(core_axis_name="c", subcore_axis_name="s")

    @functools.partial(
        pl.kernel,
        out_type=jax.ShapeDtypeStruct((N, D), jnp.float32),
        mesh=mesh)
    def gather_kernel(w_hbm, i_hbm, o_hbm):
        def body(i_vmem, o_vmem):
            pltpu.sync_copy(w_hbm.at[i_vmem.at[0]], o_vmem)

        pltpu.emit_pipeline(
            body,
            grid=(N // GW,),
            in_specs=[pl.BlockSpec((1, GW), lambda i: (0, i))],
            out_specs=[pl.BlockSpec((GW, D), lambda i: (i, 0))],
            core_axis_name=("c", "s"),
            dimension_semantics=(pltpu.PARALLEL,),
        )(i_hbm, o_hbm)

    return gather_kernel(w, idx)


def _add_body(a_ref, b_ref, o_ref):
    o_ref[...] = a_ref[...] + b_ref[...]


def _tc_add(a, b):
    blk = pl.BlockSpec((2048, D), lambda i: (i, 0))
    return pl.pallas_call(
        _add_body,
        grid=(N // 2048,),
        in_specs=[blk, blk],
        out_specs=blk,
        out_shape=jax.ShapeDtypeStruct((N, D), jnp.float32),
    )(a, b)


def kernel(z, W0, W1, W2, W3):
    Ws = (W0, W1, W2, W3)
    wts = tuple(jnp.transpose(w) for w in Ws)

    codes = []
    residuals = [z]

    c2d = _tc_layer(z, None, None, wts[0], mode=0)
    codes.append(c2d.reshape(N))
    g = _sc_gather(Ws[0], codes[0].reshape(1, N))

    q = None
    for l in range(1, NUM_LAYERS):
        tail = "spill3" if l == 2 else "fast"
        if l == 1:
            r2d, c2d = _tc_layer(z, None, g, wts[l], mode=1, tail=tail)
            qnew = g
        else:
            qnew, r2d, c2d = _tc_layer(z, q, g, wts[l], mode=2, tail=tail)
        residuals.append(r2d)
        codes.append(c2d.reshape(N))
        g = _sc_gather(Ws[l], codes[l].reshape(1, N))
        q = qnew

    quantized = _tc_add(q, g)
    return (tuple(codes), quantized, tuple(residuals))
